# baseline (speedup 1.0000x reference)
import functools

import jax
import jax.numpy as jnp
from jax import lax
from jax.experimental import pallas as pl
from jax.experimental.pallas import tpu as pltpu
from jax.experimental.pallas import tpu_sc as plsc

_K = 16
_G = 8
_NW = 32
_CH = 128
_CV = 125
_NBUF = 8
_PF = 4


def _assign_coarse_body(nodes_ref, w1_ref, b1_ref, w2_ref, b2_ref,
                        assign_ref, coarse_ref):
    n, d = nodes_ref.shape
    x = nodes_ref[...]
    h = jnp.maximum(
        jnp.dot(x, w1_ref[...], preferred_element_type=jnp.float32)
        + b1_ref[...], 0.0)
    logits = (jnp.dot(h, w2_ref[...], preferred_element_type=jnp.float32)
              + b2_ref[...])
    m = jnp.max(logits, axis=-1, keepdims=True)
    e = jnp.exp(logits - m)
    a = e / jnp.sum(e, axis=-1, keepdims=True)
    assign_ref[...] = a
    per = n // _G
    a3 = a.reshape(_G, per, _K)
    x3 = x.reshape(_G, per, d)
    for g in range(_G):
        coarse_ref[g * _K:(g + 1) * _K, :] = lax.dot_general(
            a3[g], x3[g], (((0,), (0,)), ((), ())),
            preferred_element_type=jnp.float32)


def _make_edge_kernel(n_nodes, n_edges):
    epw = n_edges // _NW
    nch = epw // _CV
    assert n_edges % _NW == 0 and epw % _CV == 0 and nch % _NBUF == 0
    gpc = 2
    mrows = gpc * n_nodes + 8
    zrows = gpc * n_nodes // 16
    half = n_nodes // 2
    piece = half // 5
    prow = piece * _K // 128
    gr = half * _K // 128

    mesh = plsc.VectorSubcoreMesh(core_axis_name="c", subcore_axis_name="s")

    @functools.partial(
        pl.kernel,
        out_type=jax.ShapeDtypeStruct((2 * gpc, n_nodes * _K // 128, 128),
                                      jnp.float32),
        mesh=mesh,
        compiler_params=pltpu.CompilerParams(use_tc_tiling_on_sc=False),
        scratch_types=[
            pltpu.VMEM((nch, _CH), jnp.int32),
            pltpu.VMEM((nch, _CH), jnp.int32),
            pltpu.VMEM((_NBUF, _CH, _K), jnp.float32),
            pltpu.VMEM((zrows, _K), jnp.float32),
            pltpu.VMEM((piece, _K), jnp.float32),
            pltpu.VMEM((prow, 128), jnp.float32),
            pltpu.VMEM_SHARED((mrows, _K), jnp.float32),
        ] + [pltpu.SemaphoreType.DMA] * (2 * _NBUF),
    )
    def edge_kernel(assign_hbm, sidx_hbm, ridx_hbm, mpk_hbm,
                    sidx, ridx, rbuf, zbuf, pin, pout, msh, *sems):
        gsem = sems[:_NBUF]
        asem = sems[_NBUF:]
        g0, g1 = gsem[0], gsem[1]
        c = lax.axis_index("c")
        s = lax.axis_index("s")
        wid = c * 16 + s

        cp_si = pltpu.async_copy(sidx_hbm.at[pl.ds(wid * nch, nch)], sidx, g0)
        cp_ri = pltpu.async_copy(ridx_hbm.at[pl.ds(wid * nch, nch)], ridx, g1)

        zero = jnp.zeros((_K,), jnp.float32)

        def zfill(r, _):
            zbuf[r] = zero
            return 0

        lax.fori_loop(0, zrows, zfill, 0)
        pltpu.sync_copy(zbuf, msh.at[pl.ds(s * zrows, zrows)])
        cp_si.wait()
        cp_ri.wait()
        plsc.subcore_barrier()

        def g_issue(j, b):
            pltpu.async_copy(assign_hbm.at[ridx.at[j]], rbuf.at[b], gsem[b])

        def g_wait(b):
            pltpu.make_async_copy(assign_hbm.at[ridx.at[0]], rbuf.at[b],
                                  gsem[b]).wait()

        def a_issue(j, b):
            pltpu.async_copy(rbuf.at[b], msh.at[sidx.at[j]], asem[b],
                             add=True)

        def a_wait(b):
            pltpu.make_async_copy(rbuf.at[b], msh.at[sidx.at[0]],
                                  asem[b]).wait()

        for b in range(_PF):
            g_issue(b, b)

        def group(q, _):
            for b in range(_NBUF):
                j = q * _NBUF + b
                g_wait(b)
                a_issue(j, b)
                jr = j + _PF
                br = (b + _PF) % _NBUF

                @pl.when(jnp.logical_and(jr < nch, jr >= _NBUF))
                def _():
                    a_wait(br)

                @pl.when(jr < nch)
                def _():
                    g_issue(jr, br)

            return 0

        lax.fori_loop(0, nch // _NBUF, group, 0)
        for b in range(_NBUF):
            a_wait(b)
        plsc.subcore_barrier()

        @pl.when(s < 4)
        def _():
            gl = s // 2
            h = s % 2
            mbase = gl * n_nodes + h * half

            def one_piece(p, _):
                pltpu.sync_copy(msh.at[pl.ds(mbase + p * piece, piece)], pin)

                def rp(r, _):
                    for i2 in range(8):
                        pout[r, pl.ds(i2 * _K, _K)] = pin[r * 8 + i2]
                    return 0

                lax.fori_loop(0, prow, rp, 0)
                pltpu.sync_copy(
                    pout,
                    mpk_hbm.at[c * gpc + gl, pl.ds(h * gr + p * prow, prow)])
                return 0

            lax.fori_loop(0, 5, one_piece, 0)

    return edge_kernel


def _contract_body(apk_ref, mpk_ref, out_ref):
    p = lax.dot_general(apk_ref[...], mpk_ref[0],
                        (((0,), (0,)), ((), ())),
                        preferred_element_type=jnp.float32)
    acc = p[0:_K, 0:_K]
    for a in range(1, 8):
        acc = acc + p[_K * a:_K * (a + 1), _K * a:_K * (a + 1)]
    out_ref[0] = acc


def kernel(nodes, senders, receivers, n_node, n_edge, W1, b1, W2, b2):
    n, d = nodes.shape
    e = senders.shape[0]
    g, k = _G, _K

    assignments, coarse_nodes = pl.pallas_call(
        _assign_coarse_body,
        out_shape=(jax.ShapeDtypeStruct((n, k), jnp.float32),
                   jax.ShapeDtypeStruct((g * k, d), jnp.float32)),
    )(nodes, W1, b1.reshape(1, -1), W2, b2.reshape(1, -1))

    epg = e // g
    gid = jnp.arange(e, dtype=jnp.int32) // epg
    sloc = senders.astype(jnp.int32) + (gid % 2) * n
    r32 = receivers.astype(jnp.int32)

    ec = e // 2
    rows = ec // _CV
    trash = jnp.int32(2 * n)

    def prep(x, pad_val):
        return jnp.pad(x.reshape(rows, _CV), ((0, 0), (0, _CH - _CV)),
                       constant_values=pad_val)

    ek = _make_edge_kernel(n, ec)
    m0 = ek(assignments, prep(sloc[:ec], trash), prep(r32[:ec], 0))
    m1 = ek(assignments, prep(sloc[ec:], trash), prep(r32[ec:], 0))

    apk = assignments.reshape(n * k // 128, 128)
    contract = pl.pallas_call(
        _contract_body,
        grid=(g // 2,),
        in_specs=[
            pl.BlockSpec((n * k // 128, 128), lambda i: (0, 0)),
            pl.BlockSpec((1, n * k // 128, 128), lambda i: (i, 0, 0)),
        ],
        out_specs=pl.BlockSpec((1, k, k), lambda i: (i, 0, 0)),
        out_shape=jax.ShapeDtypeStruct((g // 2, k, k), jnp.float32),
    )
    adj0 = contract(apk, m0)
    adj1 = contract(apk, m1)
    c_edge_weights = jnp.concatenate([adj0, adj1], axis=0).reshape(
        g * k * k, 1)

    ar = jnp.arange(k * k, dtype=jnp.int32)
    offs = jnp.arange(g, dtype=jnp.int32)[:, None] * k
    c_senders = ((ar // k)[None, :] + offs).reshape(-1)
    c_receivers = ((ar % k)[None, :] + offs).reshape(-1)
    c_n_node = jnp.full((g,), k, dtype=jnp.int32)
    c_n_edge = jnp.full((g,), k * k, dtype=jnp.int32)
    return (coarse_nodes, c_senders, c_receivers, c_edge_weights,
            c_n_node, c_n_edge, assignments)


# speedup vs baseline: 1.0265x; 1.0265x over previous
import functools

import jax
import jax.numpy as jnp
from jax import lax
from jax.experimental import pallas as pl
from jax.experimental.pallas import tpu as pltpu
from jax.experimental.pallas import tpu_sc as plsc

_K = 16
_G = 8
_NW = 32
_CH = 128
_CV = 125
_NBUF = 8
_PF = 4


def _assign_coarse_body(nodes_ref, w1_ref, b1_ref, w2_ref, b2_ref,
                        assign_ref, coarse_ref):
    n, d = nodes_ref.shape
    x = nodes_ref[...]
    h = jnp.maximum(
        jnp.dot(x, w1_ref[...], preferred_element_type=jnp.float32)
        + b1_ref[...], 0.0)
    logits = (jnp.dot(h, w2_ref[...], preferred_element_type=jnp.float32)
              + b2_ref[...])
    m = jnp.max(logits, axis=-1, keepdims=True)
    e = jnp.exp(logits - m)
    a = e / jnp.sum(e, axis=-1, keepdims=True)
    assign_ref[...] = a
    per = n // _G
    a3 = a.reshape(_G, per, _K)
    x3 = x.reshape(_G, per, d)
    for g in range(_G):
        coarse_ref[g * _K:(g + 1) * _K, :] = lax.dot_general(
            a3[g], x3[g], (((0,), (0,)), ((), ())),
            preferred_element_type=jnp.float32)


def _make_edge_kernel(n_nodes, n_edges):
    epw = n_edges // _NW
    nch = epw // _CV
    assert n_edges % _NW == 0 and epw % _CV == 0 and nch % _NBUF == 0
    gpc = 2
    mrows = gpc * n_nodes + 8
    zrows = gpc * n_nodes // 16
    half = n_nodes // 2
    piece = half // 5
    prow = piece * _K // 128
    gr = half * _K // 128

    mesh = plsc.VectorSubcoreMesh(core_axis_name="c", subcore_axis_name="s")

    @functools.partial(
        pl.kernel,
        out_type=jax.ShapeDtypeStruct((2 * gpc, n_nodes * _K // 128, 128),
                                      jnp.float32),
        mesh=mesh,
        compiler_params=pltpu.CompilerParams(use_tc_tiling_on_sc=False),
        scratch_types=[
            pltpu.VMEM((nch, _CH), jnp.int32),
            pltpu.VMEM((nch, _CH), jnp.int32),
            pltpu.VMEM((_NBUF, _CH, _K), jnp.float32),
            pltpu.VMEM((zrows, _K), jnp.float32),
            pltpu.VMEM((piece, _K), jnp.float32),
            pltpu.VMEM((prow, 128), jnp.float32),
            pltpu.VMEM_SHARED((mrows, _K), jnp.float32),
        ] + [pltpu.SemaphoreType.DMA] * (2 * _NBUF),
    )
    def edge_kernel(assign_hbm, sidx_hbm, ridx_hbm, mpk_hbm,
                    sidx, ridx, rbuf, zbuf, pin, pout, msh, *sems):
        gsem = sems[:_NBUF]
        asem = sems[_NBUF:]
        g0, g1 = gsem[0], gsem[1]
        c = lax.axis_index("c")
        s = lax.axis_index("s")
        wid = c * 16 + s

        cp_si = pltpu.async_copy(sidx_hbm.at[pl.ds(wid * nch, nch)], sidx, g0)
        cp_ri = pltpu.async_copy(ridx_hbm.at[pl.ds(wid * nch, nch)], ridx, g1)

        zero = jnp.zeros((_K,), jnp.float32)

        def zfill(r, _):
            zbuf[r] = zero
            return 0

        lax.fori_loop(0, zrows, zfill, 0)
        pltpu.sync_copy(zbuf, msh.at[pl.ds(s * zrows, zrows)])
        cp_si.wait()
        cp_ri.wait()
        plsc.subcore_barrier()

        def g_issue(j, b):
            pltpu.async_copy(assign_hbm.at[ridx.at[j]], rbuf.at[b], gsem[b])

        def g_wait(b):
            pltpu.make_async_copy(assign_hbm.at[ridx.at[0]], rbuf.at[b],
                                  gsem[b]).wait()

        def a_issue(j, b):
            pltpu.async_copy(rbuf.at[b], msh.at[sidx.at[j]], asem[b],
                             add=True)

        def a_wait(b):
            pltpu.make_async_copy(rbuf.at[b], msh.at[sidx.at[0]],
                                  asem[b]).wait()

        for b in range(_PF):
            g_issue(b, b)

        def group(q, _):
            for b in range(_NBUF):
                j = q * _NBUF + b
                g_wait(b)
                a_issue(j, b)
                jr = j + _PF
                br = (b + _PF) % _NBUF

                @pl.when(jnp.logical_and(jr < nch, jr >= _NBUF))
                def _():
                    a_wait(br)

                @pl.when(jr < nch)
                def _():
                    g_issue(jr, br)

            return 0

        lax.fori_loop(0, nch // _NBUF, group, 0)
        for b in range(_NBUF):
            a_wait(b)
        plsc.subcore_barrier()

        @pl.when(s < 4)
        def _():
            gl = s // 2
            h = s % 2
            mbase = gl * n_nodes + h * half

            def one_piece(p, _):
                pltpu.sync_copy(msh.at[pl.ds(mbase + p * piece, piece)], pin)

                def rp(r, _):
                    for i2 in range(8):
                        pout[r, pl.ds(i2 * _K, _K)] = pin[r * 8 + i2]
                    return 0

                lax.fori_loop(0, prow, rp, 0)
                pltpu.sync_copy(
                    pout,
                    mpk_hbm.at[c * gpc + gl, pl.ds(h * gr + p * prow, prow)])
                return 0

            lax.fori_loop(0, 5, one_piece, 0)

    return edge_kernel


def _contract_body(apk_ref, m0_ref, m1_ref, out0_ref, out1_ref):
    apk = apk_ref[...]
    for m_ref, o_ref in ((m0_ref, out0_ref), (m1_ref, out1_ref)):
        p = lax.dot_general(apk, m_ref[0],
                            (((0,), (0,)), ((), ())),
                            preferred_element_type=jnp.float32)
        acc = p[0:_K, 0:_K]
        for a in range(1, 8):
            acc = acc + p[_K * a:_K * (a + 1), _K * a:_K * (a + 1)]
        o_ref[0] = acc


def kernel(nodes, senders, receivers, n_node, n_edge, W1, b1, W2, b2):
    n, d = nodes.shape
    e = senders.shape[0]
    g, k = _G, _K

    assignments, coarse_nodes = pl.pallas_call(
        _assign_coarse_body,
        out_shape=(jax.ShapeDtypeStruct((n, k), jnp.float32),
                   jax.ShapeDtypeStruct((g * k, d), jnp.float32)),
    )(nodes, W1, b1.reshape(1, -1), W2, b2.reshape(1, -1))

    epg = e // g
    gid = jnp.arange(e, dtype=jnp.int32) // epg
    sloc = senders.astype(jnp.int32) + (gid % 2) * n
    r32 = receivers.astype(jnp.int32)

    ec = e // 2
    rows = ec // _CV
    trash = 2 * n + (jnp.arange(rows, dtype=jnp.int32) % 8)[:, None]
    trash = jnp.broadcast_to(trash, (rows, _CH - _CV))
    zpad = jnp.zeros((rows, _CH - _CV), jnp.int32)

    def prep(x, pad_cols):
        return jnp.concatenate([x.reshape(rows, _CV), pad_cols], axis=1)

    ek = _make_edge_kernel(n, ec)
    m0 = ek(assignments, prep(sloc[:ec], trash), prep(r32[:ec], zpad))
    m1 = ek(assignments, prep(sloc[ec:], trash), prep(r32[ec:], zpad))

    apk = assignments.reshape(n * k // 128, 128)
    adj0, adj1 = pl.pallas_call(
        _contract_body,
        grid=(g // 2,),
        in_specs=[
            pl.BlockSpec((n * k // 128, 128), lambda i: (0, 0)),
            pl.BlockSpec((1, n * k // 128, 128), lambda i: (i, 0, 0)),
            pl.BlockSpec((1, n * k // 128, 128), lambda i: (i, 0, 0)),
        ],
        out_specs=[pl.BlockSpec((1, k, k), lambda i: (i, 0, 0))] * 2,
        out_shape=[jax.ShapeDtypeStruct((g // 2, k, k), jnp.float32)] * 2,
    )(apk, m0, m1)
    c_edge_weights = jnp.concatenate([adj0, adj1], axis=0).reshape(
        g * k * k, 1)

    ar = jnp.arange(k * k, dtype=jnp.int32)
    offs = jnp.arange(g, dtype=jnp.int32)[:, None] * k
    c_senders = ((ar // k)[None, :] + offs).reshape(-1)
    c_receivers = ((ar % k)[None, :] + offs).reshape(-1)
    c_n_node = jnp.full((g,), k, dtype=jnp.int32)
    c_n_edge = jnp.full((g,), k * k, dtype=jnp.int32)
    return (coarse_nodes, c_senders, c_receivers, c_edge_weights,
            c_n_node, c_n_edge, assignments)


# speedup vs baseline: 1.2591x; 1.2266x over previous
import functools

import jax
import jax.numpy as jnp
from jax import lax
from jax.experimental import pallas as pl
from jax.experimental.pallas import tpu as pltpu
from jax.experimental.pallas import tpu_sc as plsc

_K = 16
_G = 8
_NW = 32
_CH = 125
_CV = 125
_NBUF = 8
_PF = 4


def _assign_coarse_body(nodes_ref, w1_ref, b1_ref, w2_ref, b2_ref,
                        assign_ref, coarse_ref):
    n, d = nodes_ref.shape
    x = nodes_ref[...]
    h = jnp.maximum(
        jnp.dot(x, w1_ref[...], preferred_element_type=jnp.float32)
        + b1_ref[...], 0.0)
    logits = (jnp.dot(h, w2_ref[...], preferred_element_type=jnp.float32)
              + b2_ref[...])
    m = jnp.max(logits, axis=-1, keepdims=True)
    e = jnp.exp(logits - m)
    a = e / jnp.sum(e, axis=-1, keepdims=True)
    assign_ref[...] = a
    per = n // _G
    a3 = a.reshape(_G, per, _K)
    x3 = x.reshape(_G, per, d)
    for g in range(_G):
        coarse_ref[g * _K:(g + 1) * _K, :] = lax.dot_general(
            a3[g], x3[g], (((0,), (0,)), ((), ())),
            preferred_element_type=jnp.float32)


def _make_edge_kernel(n_nodes, n_edges):
    epw = n_edges // _NW
    nch = epw // _CV
    assert n_edges % _NW == 0 and epw % _CV == 0 and nch % _NBUF == 0
    gpc = 2
    mrows = gpc * n_nodes + 8
    zrows = gpc * n_nodes // 16
    half = n_nodes // 2
    piece = half // 5
    prow = piece * _K // 128
    gr = half * _K // 128

    mesh = plsc.VectorSubcoreMesh(core_axis_name="c", subcore_axis_name="s")

    @functools.partial(
        pl.kernel,
        out_type=jax.ShapeDtypeStruct((2 * gpc, n_nodes * _K // 128, 128),
                                      jnp.float32),
        mesh=mesh,
        compiler_params=pltpu.CompilerParams(use_tc_tiling_on_sc=False),
        scratch_types=[
            pltpu.VMEM((nch, _CH), jnp.int32),
            pltpu.VMEM((nch, _CH), jnp.int32),
            pltpu.VMEM((_NBUF, _CH, _K), jnp.float32),
            pltpu.VMEM((zrows, _K), jnp.float32),
            pltpu.VMEM((piece, _K), jnp.float32),
            pltpu.VMEM((prow, 128), jnp.float32),
            pltpu.VMEM_SHARED((mrows, _K), jnp.float32),
        ] + [pltpu.SemaphoreType.DMA] * (2 * _NBUF),
    )
    def edge_kernel(assign_hbm, sidx_hbm, ridx_hbm, mpk_hbm,
                    sidx, ridx, rbuf, zbuf, pin, pout, msh, *sems):
        gsem = sems[:_NBUF]
        asem = sems[_NBUF:]
        g0, g1 = gsem[0], gsem[1]
        c = lax.axis_index("c")
        s = lax.axis_index("s")
        wid = c * 16 + s

        cp_si = pltpu.async_copy(sidx_hbm.at[pl.ds(wid * nch, nch)], sidx, g0)
        cp_ri = pltpu.async_copy(ridx_hbm.at[pl.ds(wid * nch, nch)], ridx, g1)

        zero = jnp.zeros((_K,), jnp.float32)

        def zfill(r, _):
            zbuf[r] = zero
            return 0

        lax.fori_loop(0, zrows, zfill, 0)
        pltpu.sync_copy(zbuf, msh.at[pl.ds(s * zrows, zrows)])
        cp_si.wait()
        cp_ri.wait()
        plsc.subcore_barrier()

        def g_issue(j, b):
            pltpu.async_copy(assign_hbm.at[ridx.at[j]], rbuf.at[b], gsem[b])

        def g_wait(b):
            pltpu.make_async_copy(assign_hbm.at[ridx.at[0]], rbuf.at[b],
                                  gsem[b]).wait()

        def a_issue(j, b):
            pltpu.async_copy(rbuf.at[b], msh.at[sidx.at[j]], asem[b],
                             add=True)

        def a_wait(b):
            pltpu.make_async_copy(rbuf.at[b], msh.at[sidx.at[0]],
                                  asem[b]).wait()

        for b in range(_PF):
            g_issue(b, b)

        def group(q, _):
            for b in range(_NBUF):
                j = q * _NBUF + b
                g_wait(b)
                a_issue(j, b)
                jr = j + _PF
                br = (b + _PF) % _NBUF

                @pl.when(jnp.logical_and(jr < nch, jr >= _NBUF))
                def _():
                    a_wait(br)

                @pl.when(jr < nch)
                def _():
                    g_issue(jr, br)

            return 0

        lax.fori_loop(0, nch // _NBUF, group, 0)
        for b in range(_NBUF):
            a_wait(b)
        plsc.subcore_barrier()

        @pl.when(s < 4)
        def _():
            gl = s // 2
            h = s % 2
            mbase = gl * n_nodes + h * half

            def one_piece(p, _):
                pltpu.sync_copy(msh.at[pl.ds(mbase + p * piece, piece)], pin)

                def rp(r, _):
                    for i2 in range(8):
                        pout[r, pl.ds(i2 * _K, _K)] = pin[r * 8 + i2]
                    return 0

                lax.fori_loop(0, prow, rp, 0)
                pltpu.sync_copy(
                    pout,
                    mpk_hbm.at[c * gpc + gl, pl.ds(h * gr + p * prow, prow)])
                return 0

            lax.fori_loop(0, 5, one_piece, 0)

    return edge_kernel


def _contract_body(apk_ref, m0_ref, m1_ref, out0_ref, out1_ref):
    apk = apk_ref[...]
    for m_ref, o_ref in ((m0_ref, out0_ref), (m1_ref, out1_ref)):
        p = lax.dot_general(apk, m_ref[0],
                            (((0,), (0,)), ((), ())),
                            preferred_element_type=jnp.float32)
        acc = p[0:_K, 0:_K]
        for a in range(1, 8):
            acc = acc + p[_K * a:_K * (a + 1), _K * a:_K * (a + 1)]
        o_ref[0] = acc


def kernel(nodes, senders, receivers, n_node, n_edge, W1, b1, W2, b2):
    n, d = nodes.shape
    e = senders.shape[0]
    g, k = _G, _K

    assignments, coarse_nodes = pl.pallas_call(
        _assign_coarse_body,
        out_shape=(jax.ShapeDtypeStruct((n, k), jnp.float32),
                   jax.ShapeDtypeStruct((g * k, d), jnp.float32)),
    )(nodes, W1, b1.reshape(1, -1), W2, b2.reshape(1, -1))

    epg = e // g
    gid = jnp.arange(e, dtype=jnp.int32) // epg
    sloc = senders.astype(jnp.int32) + (gid % 2) * n
    r32 = receivers.astype(jnp.int32)

    ec = e // 2
    rows = ec // _CV

    def prep(x):
        return x.reshape(rows, _CV)

    ek = _make_edge_kernel(n, ec)
    m0 = ek(assignments, prep(sloc[:ec]), prep(r32[:ec]))
    m1 = ek(assignments, prep(sloc[ec:]), prep(r32[ec:]))

    apk = assignments.reshape(n * k // 128, 128)
    adj0, adj1 = pl.pallas_call(
        _contract_body,
        grid=(g // 2,),
        in_specs=[
            pl.BlockSpec((n * k // 128, 128), lambda i: (0, 0)),
            pl.BlockSpec((1, n * k // 128, 128), lambda i: (i, 0, 0)),
            pl.BlockSpec((1, n * k // 128, 128), lambda i: (i, 0, 0)),
        ],
        out_specs=[pl.BlockSpec((1, k, k), lambda i: (i, 0, 0))] * 2,
        out_shape=[jax.ShapeDtypeStruct((g // 2, k, k), jnp.float32)] * 2,
    )(apk, m0, m1)
    c_edge_weights = jnp.concatenate([adj0, adj1], axis=0).reshape(
        g * k * k, 1)

    ar = jnp.arange(k * k, dtype=jnp.int32)
    offs = jnp.arange(g, dtype=jnp.int32)[:, None] * k
    c_senders = ((ar // k)[None, :] + offs).reshape(-1)
    c_receivers = ((ar % k)[None, :] + offs).reshape(-1)
    c_n_node = jnp.full((g,), k, dtype=jnp.int32)
    c_n_edge = jnp.full((g,), k * k, dtype=jnp.int32)
    return (coarse_nodes, c_senders, c_receivers, c_edge_weights,
            c_n_node, c_n_edge, assignments)


# speedup vs baseline: 1.3941x; 1.1073x over previous
import functools

import jax
import jax.numpy as jnp
from jax import lax
from jax.experimental import pallas as pl
from jax.experimental.pallas import tpu as pltpu
from jax.experimental.pallas import tpu_sc as plsc

_K = 16
_G = 8
_NW = 32
_CH = 125
_CV = 125
_NBUF = 8
_PF = 4


def _assign_coarse_body(nodes_ref, w1_ref, b1_ref, w2_ref, b2_ref,
                        assign_ref, coarse_ref):
    n, d = nodes_ref.shape
    x = nodes_ref[...]
    h = jnp.maximum(
        jnp.dot(x, w1_ref[...], preferred_element_type=jnp.float32)
        + b1_ref[...], 0.0)
    logits = (jnp.dot(h, w2_ref[...], preferred_element_type=jnp.float32)
              + b2_ref[...])
    m = jnp.max(logits, axis=-1, keepdims=True)
    e = jnp.exp(logits - m)
    a = e / jnp.sum(e, axis=-1, keepdims=True)
    assign_ref[...] = a
    per = n // _G
    a3 = a.reshape(_G, per, _K)
    x3 = x.reshape(_G, per, d)
    for g in range(_G):
        coarse_ref[g * _K:(g + 1) * _K, :] = lax.dot_general(
            a3[g], x3[g], (((0,), (0,)), ((), ())),
            preferred_element_type=jnp.float32)


def _make_edge_kernel(n_nodes, n_edges):
    epw = n_edges // _NW
    nch = epw // _CV
    assert n_edges % _NW == 0 and epw % _CV == 0 and nch % _NBUF == 0
    gpc = 2
    mrows = gpc * n_nodes + 8
    zrows = gpc * n_nodes // 16
    half = n_nodes // 2
    piece = half // 5
    prow = piece * _K // 128
    gr = half * _K // 128

    mesh = plsc.VectorSubcoreMesh(core_axis_name="c", subcore_axis_name="s")

    @functools.partial(
        pl.kernel,
        out_type=jax.ShapeDtypeStruct((2 * gpc, n_nodes * _K // 128, 128),
                                      jnp.float32),
        mesh=mesh,
        compiler_params=pltpu.CompilerParams(use_tc_tiling_on_sc=False),
        scratch_types=[
            pltpu.VMEM((nch, _CH), jnp.int32),
            pltpu.VMEM((nch, _CH), jnp.int32),
            pltpu.VMEM((_NBUF, _CH, _K), jnp.float32),
            pltpu.VMEM((zrows, _K), jnp.float32),
            pltpu.VMEM((piece, _K), jnp.float32),
            pltpu.VMEM((prow, 128), jnp.float32),
            pltpu.VMEM_SHARED((mrows, _K), jnp.float32),
        ] + [pltpu.SemaphoreType.DMA] * (2 * _NBUF),
    )
    def edge_kernel(assign_hbm, sidx_hbm, ridx_hbm, mpk_hbm,
                    sidx, ridx, rbuf, zbuf, pin, pout, msh, *sems):
        gsem = sems[:_NBUF]
        asem = sems[_NBUF:]
        g0, g1 = gsem[0], gsem[1]
        c = lax.axis_index("c")
        s = lax.axis_index("s")
        wid = c * 16 + s

        cp_si = pltpu.async_copy(sidx_hbm.at[pl.ds(wid * nch, nch)], sidx, g0)
        cp_ri = pltpu.async_copy(ridx_hbm.at[pl.ds(wid * nch, nch)], ridx, g1)

        zero = jnp.zeros((_K,), jnp.float32)

        def zfill(r, _):
            zbuf[r] = zero
            return 0

        lax.fori_loop(0, zrows, zfill, 0)
        pltpu.sync_copy(zbuf, msh.at[pl.ds(s * zrows, zrows)])
        cp_si.wait()
        cp_ri.wait()
        plsc.subcore_barrier()

        def g_issue(j, b):
            pltpu.async_copy(assign_hbm.at[ridx.at[j]], rbuf.at[b], gsem[b])

        def g_wait(b):
            pltpu.make_async_copy(assign_hbm.at[ridx.at[0]], rbuf.at[b],
                                  gsem[b]).wait()

        def a_issue(j, b):
            pltpu.async_copy(rbuf.at[b], msh.at[sidx.at[j]], asem[b],
                             add=True)

        def a_wait(b):
            pltpu.make_async_copy(rbuf.at[b], msh.at[sidx.at[0]],
                                  asem[b]).wait()

        for b in range(_PF):
            g_issue(b, b)

        def group(q, _):
            for b in range(_NBUF):
                j = q * _NBUF + b
                g_wait(b)
                a_issue(j, b)
                jr = j + _PF
                br = (b + _PF) % _NBUF

                @pl.when(jnp.logical_and(jr < nch, jr >= _NBUF))
                def _():
                    a_wait(br)

                @pl.when(jr < nch)
                def _():
                    g_issue(jr, br)

            return 0

        lax.fori_loop(0, nch // _NBUF, group, 0)
        for b in range(_NBUF):
            a_wait(b)
        plsc.subcore_barrier()

        npiece = gpc * n_nodes // piece

        def do_piece(q):
            gl = q // (npiece // gpc)
            po = q % (npiece // gpc)
            pltpu.sync_copy(msh.at[pl.ds(gl * n_nodes + po * piece, piece)],
                            pin)

            def rp(r, _):
                for i2 in range(8):
                    pout[r, pl.ds(i2 * _K, _K)] = pin[r * 8 + i2]
                return 0

            lax.fori_loop(0, prow, rp, 0)
            pltpu.sync_copy(
                pout,
                mpk_hbm.at[c * gpc + gl, pl.ds(po * prow, prow)])

        do_piece(s)

        @pl.when(s < npiece - 16)
        def _():
            do_piece(s + 16)

    return edge_kernel


def _contract_body(apk_ref, m0_ref, m1_ref, out0_ref, out1_ref):
    apk = apk_ref[...]
    for m_ref, o_ref in ((m0_ref, out0_ref), (m1_ref, out1_ref)):
        p = lax.dot_general(apk, m_ref[0],
                            (((0,), (0,)), ((), ())),
                            preferred_element_type=jnp.float32)
        acc = p[0:_K, 0:_K]
        for a in range(1, 8):
            acc = acc + p[_K * a:_K * (a + 1), _K * a:_K * (a + 1)]
        o_ref[0] = acc


def kernel(nodes, senders, receivers, n_node, n_edge, W1, b1, W2, b2):
    n, d = nodes.shape
    e = senders.shape[0]
    g, k = _G, _K

    assignments, coarse_nodes = pl.pallas_call(
        _assign_coarse_body,
        out_shape=(jax.ShapeDtypeStruct((n, k), jnp.float32),
                   jax.ShapeDtypeStruct((g * k, d), jnp.float32)),
    )(nodes, W1, b1.reshape(1, -1), W2, b2.reshape(1, -1))

    epg = e // g
    gid = jnp.arange(e, dtype=jnp.int32) // epg
    sloc = senders.astype(jnp.int32) + (gid % 2) * n
    r32 = receivers.astype(jnp.int32)

    ec = e // 2
    rows = ec // _CV

    def prep(x):
        return x.reshape(rows, _CV)

    ek = _make_edge_kernel(n, ec)
    m0 = ek(assignments, prep(sloc[:ec]), prep(r32[:ec]))
    m1 = ek(assignments, prep(sloc[ec:]), prep(r32[ec:]))

    apk = assignments.reshape(n * k // 128, 128)
    adj0, adj1 = pl.pallas_call(
        _contract_body,
        grid=(g // 2,),
        in_specs=[
            pl.BlockSpec((n * k // 128, 128), lambda i: (0, 0)),
            pl.BlockSpec((1, n * k // 128, 128), lambda i: (i, 0, 0)),
            pl.BlockSpec((1, n * k // 128, 128), lambda i: (i, 0, 0)),
        ],
        out_specs=[pl.BlockSpec((1, k, k), lambda i: (i, 0, 0))] * 2,
        out_shape=[jax.ShapeDtypeStruct((g // 2, k, k), jnp.float32)] * 2,
    )(apk, m0, m1)
    c_edge_weights = jnp.concatenate([adj0, adj1], axis=0).reshape(
        g * k * k, 1)

    ar = jnp.arange(k * k, dtype=jnp.int32)
    offs = jnp.arange(g, dtype=jnp.int32)[:, None] * k
    c_senders = ((ar // k)[None, :] + offs).reshape(-1)
    c_receivers = ((ar % k)[None, :] + offs).reshape(-1)
    c_n_node = jnp.full((g,), k, dtype=jnp.int32)
    c_n_edge = jnp.full((g,), k * k, dtype=jnp.int32)
    return (coarse_nodes, c_senders, c_receivers, c_edge_weights,
            c_n_node, c_n_edge, assignments)
